# baseline (speedup 1.0000x reference)
import functools

import jax
import jax.numpy as jnp
from jax import lax
from jax.experimental import pallas as pl
from jax.experimental.pallas import tpu as pltpu
from jax.experimental.pallas import tpu_sc as plsc

N = 10000
D = 256
NPAD = 10240
NC, NS = 2, 16
NW = NC * NS
RPT = NPAD // NW
EB = 128
BE = 2048

_DNUMS = lax.GatherDimensionNumbers(
    offset_dims=(), collapsed_slice_dims=(0,), start_index_map=(0,))


def _mesh():
    return plsc.VectorSubcoreMesh(
        core_axis_name="c", subcore_axis_name="s", num_cores=NC,
        num_subcores=NS)


def _wid():
    return lax.axis_index("s") * NC + lax.axis_index("c")


def _splat(v, e):
    return lax.gather(v, jnp.full((16, 1), e, jnp.int32), _DNUMS, (1,),
                      mode=lax.GatherScatterMode.PROMISE_IN_BOUNDS)


def _csum_lanes(m):
    c = m
    for k in (1, 2, 4, 8, 16, 32, 64):
        c = c + jnp.concatenate(
            [jnp.zeros((16, k), jnp.int32), c[:, : 128 - k]], axis=1)
    return c


def _csum_rows(x):
    c = x
    for k in (1, 2, 4, 8):
        c = c + jnp.concatenate(
            [jnp.zeros((k, 128), jnp.int32), c[: 16 - k, :]], axis=0)
    return c


def _bucket_pos_tc(dst3, cap):
    nbk = dst3.shape[0]

    def body(d_ref, pos_ref, cnt_ref, carry):
        i = pl.program_id(0)

        @pl.when(i == 0)
        def _():
            for t in range(NW):
                carry[t] = 0

        d = d_ref[0]
        bkt = d // RPT
        pos = jnp.zeros((16, 128), jnp.int32)
        for t in range(NW):
            m = jnp.where(bkt == t, 1, 0)
            c1 = _csum_lanes(m)
            rowtot = jnp.broadcast_to(c1[:, 127:128], (16, 128))
            rowoff = _csum_rows(rowtot) - rowtot
            rank = c1 - m + rowoff
            base = carry[t] + t * cap
            pos = pos + m * (base + rank)
            carry[t] = carry[t] + jnp.sum(m)

        pos_ref[0] = pos

        @pl.when(i == nbk - 1)
        def _():
            for t in range(NW):
                cnt_ref[t, :] = jnp.full((128,), carry[t], jnp.int32)

    return pl.pallas_call(
        body,
        grid=(nbk,),
        in_specs=[pl.BlockSpec((1, 16, 128), lambda i: (i, 0, 0))],
        out_specs=[pl.BlockSpec((1, 16, 128), lambda i: (i, 0, 0)),
                   pl.BlockSpec((NW, 128), lambda i: (0, 0))],
        out_shape=[jax.ShapeDtypeStruct((nbk, 16, 128), jnp.int32),
                   jax.ShapeDtypeStruct((NW, 128), jnp.int32)],
        scratch_shapes=[pltpu.SMEM((NW,), jnp.int32)],
    )(dst3)


def _scatter_edges_sc(dst, src, ew, pos):
    e_pad = dst.shape[0]
    cap = e_pad
    epw = e_pad // NW
    nb = epw // EB

    @functools.partial(
        pl.kernel,
        out_type=(
            jax.ShapeDtypeStruct((NW * cap,), jnp.int32),
            jax.ShapeDtypeStruct((NW * cap,), jnp.int32),
            jax.ShapeDtypeStruct((NW * cap,), jnp.float32),
        ),
        mesh=_mesh(),
        scratch_types=[
            pltpu.VMEM((EB,), jnp.int32),
            pltpu.VMEM((EB,), jnp.int32),
            pltpu.VMEM((EB,), jnp.float32),
            pltpu.VMEM((EB,), jnp.int32),
        ],
    )
    def scat(dst_hbm, src_hbm, ew_hbm, pos_hbm, bd, bs, bw,
             dbuf, sbuf, wbuf, pbuf):
        ebase = _wid() * epw

        def batch(b, _):
            off = pl.multiple_of(ebase + b * EB, 16)
            pltpu.sync_copy(dst_hbm.at[pl.ds(off, EB)], dbuf)
            pltpu.sync_copy(src_hbm.at[pl.ds(off, EB)], sbuf)
            pltpu.sync_copy(ew_hbm.at[pl.ds(off, EB)], wbuf)
            pltpu.sync_copy(pos_hbm.at[pl.ds(off, EB)], pbuf)
            pltpu.sync_copy(dbuf, bd.at[pbuf])
            pltpu.sync_copy(sbuf, bs.at[pbuf])
            pltpu.sync_copy(wbuf, bw.at[pbuf])
            return 0
        lax.fori_loop(0, nb, batch, 0)

    return scat(dst, src, ew, pos)


def _spmm_sc(S, bd, bs, bw, counts):
    cap = bd.shape[0] // NW

    @functools.partial(
        pl.kernel,
        out_type=jax.ShapeDtypeStruct((NPAD, D), jnp.float32),
        mesh=_mesh(),
        scratch_types=[
            pltpu.VMEM((RPT, D), jnp.float32),
            pltpu.VMEM((EB, D), jnp.float32),
            pltpu.VMEM((EB,), jnp.int32),
            pltpu.VMEM((EB,), jnp.int32),
            pltpu.VMEM((EB,), jnp.float32),
            pltpu.VMEM((16,), jnp.int32),
            pltpu.SemaphoreType.DMA,
        ],
    )
    def spmm(s_hbm, bd_hbm, bs_hbm, bw_hbm, cnt_hbm, out_hbm,
             acc, rows, dsts, srcs, ews, cbuf, sem):
        t = _wid()
        lo = t * RPT
        base = t * cap

        pltpu.sync_copy(cnt_hbm.at[t, pl.ds(0, 16)], cbuf)
        cnt = cbuf[pl.ds(0, 16)][0]
        nb = (cnt + EB - 1) // EB

        def zrow(r, _):
            for g in range(D // 16):
                acc[r, pl.ds(g * 16, 16)] = jnp.zeros((16,), jnp.float32)
            return 0
        lax.fori_loop(0, RPT, zrow, 0)

        lane = lax.iota(jnp.int32, 16)
        lo_v = jnp.full((16,), lo, jnp.int32)
        zero_v = jnp.zeros((16,), jnp.float32)
        zero_iv = jnp.zeros((16,), jnp.int32)

        def batch(b, _):
            eb0 = pl.multiple_of(base + b * EB, 16)
            pltpu.sync_copy(bd_hbm.at[pl.ds(eb0, EB)], dsts)
            pltpu.sync_copy(bs_hbm.at[pl.ds(eb0, EB)], srcs)
            pltpu.sync_copy(bw_hbm.at[pl.ds(eb0, EB)], ews)

            k0 = b * EB
            cnt_v = jnp.full((16,), cnt, jnp.int32)
            for g in range(EB // 16):
                sl = pl.ds(g * 16, 16)
                valid = (jnp.full((16,), k0 + g * 16, jnp.int32)
                         + lane) < cnt_v
                dsts[sl] = jnp.where(valid, dsts[sl], lo_v)
                srcs[sl] = jnp.where(valid, srcs[sl], zero_iv)
                ews[sl] = jnp.where(valid, ews[sl], zero_v)

            pltpu.async_copy(s_hbm.at[srcs], rows, sem).wait()

            def grp(g, _):
                sl = pl.ds(g * 16, 16)
                ld16 = dsts[sl] - lo_v
                w16 = ews[sl]
                for e in range(16):
                    ld = ld16[e]
                    wv = _splat(w16, e)
                    for gg in range(D // 16):
                        fsl = pl.ds(gg * 16, 16)
                        acc[ld, fsl] = (acc[ld, fsl]
                                        + wv * rows[g * 16 + e, fsl])
                return 0
            lax.fori_loop(0, EB // 16, grp, 0)
            return 0
        lax.fori_loop(0, nb, batch, 0)

        pltpu.sync_copy(acc, out_hbm.at[pl.ds(lo, RPT)])

    return spmm(S, bd, bs, bw, counts)


def _p_spec(rb):
    return pl.BlockSpec((rb, D), lambda i: (i, 0))


def _act(p_ref, b_ref):
    x = p_ref[...] + b_ref[0]
    return jnp.maximum(x, 0.25 * x)


def _matmul(x, w, rb):
    m = x.shape[0]

    def body(x_ref, w_ref, o_ref):
        o_ref[...] = jnp.dot(x_ref[...], w_ref[...],
                             preferred_element_type=jnp.float32)

    return pl.pallas_call(
        body,
        grid=(m // rb,),
        in_specs=[pl.BlockSpec((rb, D), lambda i: (i, 0)),
                  pl.BlockSpec((D, D), lambda i: (0, 0))],
        out_specs=pl.BlockSpec((rb, D), lambda i: (i, 0)),
        out_shape=jax.ShapeDtypeStruct((m, D), jnp.float32),
    )(x, w)


def _act_matmul(p, b, w):
    rb = 1000

    def body(p_ref, b_ref, w_ref, o_ref):
        o_ref[...] = jnp.dot(_act(p_ref, b_ref), w_ref[...],
                             preferred_element_type=jnp.float32)

    return pl.pallas_call(
        body,
        grid=(N // rb,),
        in_specs=[_p_spec(rb),
                  pl.BlockSpec((1, D), lambda i: (0, 0)),
                  pl.BlockSpec((D, D), lambda i: (0, 0))],
        out_specs=pl.BlockSpec((rb, D), lambda i: (i, 0)),
        out_shape=jax.ShapeDtypeStruct((N, D), jnp.float32),
    )(p, b, w)


def _act_only(p, b):
    rb = 1000

    def body(p_ref, b_ref, o_ref):
        o_ref[...] = _act(p_ref, b_ref)

    return pl.pallas_call(
        body,
        grid=(N // rb,),
        in_specs=[_p_spec(rb), pl.BlockSpec((1, D), lambda i: (0, 0))],
        out_specs=pl.BlockSpec((rb, D), lambda i: (i, 0)),
        out_shape=jax.ShapeDtypeStruct((N, D), jnp.float32),
    )(p, b)


def _matmul_bt(a, hd):
    m, t = a.shape[0], hd.shape[0]
    mb, tb = 1000, 1024

    def body(a_ref, h_ref, o_ref):
        o_ref[...] = lax.dot_general(
            a_ref[...], h_ref[...], (((1,), (1,)), ((), ())),
            preferred_element_type=jnp.float32)

    return pl.pallas_call(
        body,
        grid=(m // mb, pl.cdiv(t, tb)),
        in_specs=[pl.BlockSpec((mb, D), lambda i, j: (i, 0)),
                  pl.BlockSpec((tb, D), lambda i, j: (j, 0))],
        out_specs=pl.BlockSpec((mb, tb), lambda i, j: (i, j)),
        out_shape=jax.ShapeDtypeStruct((m, t), jnp.float32),
    )(a, hd)


def kernel(H, edge_index, edge_weight, W1, b1, W2, b2, W3, b3, train_W,
           drug_num, target_num):
    e = edge_weight.shape[0]
    epw = pl.cdiv(pl.cdiv(e, NW), EB) * EB
    e_pad = epw * NW
    pad = e_pad - e
    dst = jnp.concatenate([edge_index[0], jnp.full((pad,), N, jnp.int32)])
    src = jnp.concatenate([edge_index[1], jnp.zeros((pad,), jnp.int32)])
    ew = jnp.concatenate([edge_weight, jnp.zeros((pad,), jnp.float32)])

    b1r = b1.reshape(1, D)
    b2r = b2.reshape(1, D)
    b3r = b3.reshape(1, D)

    pos3, counts = _bucket_pos_tc(dst.reshape(e_pad // BE, 16, 128), e_pad)
    bd, bs, bw = _scatter_edges_sc(dst, src, ew, pos3.reshape(e_pad))

    s1 = _matmul(H, W1, 1000)
    p1 = _spmm_sc(s1, bd, bs, bw, counts)
    s2 = _act_matmul(p1, b1r, W2)
    p2 = _spmm_sc(s2, bd, bs, bw, counts)
    s3 = _act_matmul(p2, b2r, W3)
    p3 = _spmm_sc(s3, bd, bs, bw, counts)
    h3 = _act_only(p3, b3r)

    hr = lax.dynamic_slice_in_dim(h3, 0, 4000, axis=0)
    hd = lax.dynamic_slice_in_dim(h3, drug_num, 6000, axis=0)
    a = _matmul(hr, train_W, 1000)
    return _matmul_bt(a, hd)
